# baseline (speedup 1.0000x reference)
import functools

import jax
import jax.numpy as jnp
from jax import lax
from jax.experimental import pallas as pl
from jax.experimental.pallas import tpu as pltpu
from jax.experimental.pallas import tpu_sc as plsc

_B = 16384
_L = 9
_K = 10
_NCORES = 2
_NSUB = 16
_NW = _NCORES * _NSUB
_BC = _B // _NW
_NBC = _BC * _L
_CH = 128
_UC = _BC // _CH
_NCH = _NBC // _CH
_UCP = 8
_NCHP = 40

_mesh = plsc.VectorSubcoreMesh(
    core_axis_name="c", subcore_axis_name="s",
    num_cores=_NCORES, num_subcores=_NSUB,
)


@functools.partial(
    pl.kernel,
    out_type=(
        jax.ShapeDtypeStruct((_B, _K), jnp.float32),
        jax.ShapeDtypeStruct((_L, _B, _K), jnp.float32),
    ),
    mesh=_mesh,
    scratch_types=[
        pltpu.VMEM((_UCP, _CH), jnp.int32),
        pltpu.VMEM((_NCHP, _CH), jnp.int32),
        pltpu.VMEM((_BC, _K), jnp.float32),
        pltpu.VMEM((_NBC, _K), jnp.float32),
        pltpu.SemaphoreType.DMA,
        pltpu.SemaphoreType.DMA,
    ],
    compiler_params=pltpu.CompilerParams(use_tc_tiling_on_sc=False),
)
def _sc_gather(user_hbm, nearby_hbm, table_hbm, urows_hbm, nbrows_hbm,
               uidx_v, nbidx_v, urows_v, nbrows_v, sem_u, sem_nb):
    wid = lax.axis_index("s") * _NCORES + lax.axis_index("c")

    pltpu.sync_copy(user_hbm.at[pl.ds(wid * _UCP, _UCP)], uidx_v)
    pltpu.sync_copy(nearby_hbm.at[pl.ds(wid * _NCHP, _NCHP)], nbidx_v)

    for c in range(_UC):
        pltpu.async_copy(table_hbm.at[uidx_v.at[c]],
                         urows_v.at[pl.ds(c * _CH, _CH)], sem_u).wait()

    def _fire(c, carry):
        pltpu.async_copy(table_hbm.at[nbidx_v.at[c]],
                         nbrows_v.at[pl.ds(c * _CH, _CH)], sem_nb).wait()
        return carry
    lax.fori_loop(0, _NCH, _fire, 0)

    pltpu.sync_copy(urows_v, urows_hbm.at[pl.ds(wid * _BC, _BC)])
    for l in range(_L):
        pltpu.sync_copy(nbrows_v.at[pl.ds(l * _BC, _BC)],
                        nbrows_hbm.at[l, pl.ds(wid * _BC, _BC)])


_BS = 1024
_NBLK = _B // _BS


def _tc_body(user_ref, nbt_ref, urows_ref, nbrows_ref, out_ref):
    nbsum = jnp.zeros((_BS, _K), jnp.float32)
    for l in range(_L):
        m = (nbt_ref[l, :] != 0).astype(jnp.float32)
        nbsum = nbsum + nbrows_ref[l] * m[:, None]
    u = urows_ref[...]
    pred = jnp.sum(u * nbsum, axis=1)
    um = (user_ref[0, 0, :] != 0).astype(jnp.float32)
    out_ref[0, 0, :] = pred * um


_tc_compute = pl.pallas_call(
    _tc_body,
    grid=(_NBLK,),
    in_specs=[
        pl.BlockSpec((1, 1, _BS), lambda i: (i, 0, 0)),
        pl.BlockSpec((_L, _BS), lambda i: (0, i)),
        pl.BlockSpec((_BS, _K), lambda i: (i, 0)),
        pl.BlockSpec((_L, _BS, _K), lambda i: (0, i, 0)),
    ],
    out_specs=pl.BlockSpec((1, 1, _BS), lambda i: (i, 0, 0)),
    out_shape=jax.ShapeDtypeStruct((_NBLK, 1, _BS), jnp.float32),
)


@jax.jit
def kernel(user, nearby, table):
    u32 = user.astype(jnp.int32)
    nb32 = nearby.astype(jnp.int32)

    uidx = u32.reshape(_NW, _UC, _CH)
    uidx = jnp.pad(uidx, ((0, 0), (0, _UCP - _UC), (0, 0)))
    uidx = uidx.reshape(_NW * _UCP, _CH)

    nbt = nb32.T
    nbidx = nbt.reshape(_L, _NW, _BC).transpose(1, 0, 2).reshape(
        _NW, _NCH, _CH)
    nbidx = jnp.pad(nbidx, ((0, 0), (0, _NCHP - _NCH), (0, 0)))
    nbidx = nbidx.reshape(_NW * _NCHP, _CH)

    urows, nbrows = _sc_gather(uidx, nbidx, table)
    preds = _tc_compute(u32.reshape(_NBLK, 1, _BS), nbt, urows, nbrows)
    return preds.reshape(_B)


# speedup vs baseline: 1.0922x; 1.0922x over previous
import functools

import jax
import jax.numpy as jnp
from jax import lax
from jax.experimental import pallas as pl
from jax.experimental.pallas import tpu as pltpu
from jax.experimental.pallas import tpu_sc as plsc

_B = 16384
_L = 9
_K = 10
_KP = 16
_NROWS = 1000064
_ZROW = 1000001
_NCORES = 2
_NSUB = 16
_NW = _NCORES * _NSUB
_BC = _B // _NW
_NBC = _BC * _L
_CH = 128
_UC = _BC // _CH
_NCH = _NBC // _CH
_UCP = 8
_NCHP = 40
_PR = _B * _KP // 128
_PRT = _PR // _NW

_mesh = plsc.VectorSubcoreMesh(
    core_axis_name="c", subcore_axis_name="s",
    num_cores=_NCORES, num_subcores=_NSUB,
)


@functools.partial(
    pl.kernel,
    out_type=jax.ShapeDtypeStruct((_PR, 128), jnp.float32),
    mesh=_mesh,
    scratch_types=[
        pltpu.VMEM((_UCP, _CH), jnp.int32),
        pltpu.VMEM((_NCHP, _CH), jnp.int32),
        pltpu.VMEM((_BC, _KP), jnp.float32),
        pltpu.VMEM((_NBC, _KP), jnp.float32),
        pltpu.VMEM((_PRT, 128), jnp.float32),
        pltpu.SemaphoreType.DMA,
        pltpu.SemaphoreType.DMA,
    ],
    compiler_params=pltpu.CompilerParams(use_tc_tiling_on_sc=False),
)
def _place_sc(user_hbm, nearby_hbm, table_hbm, psum_hbm,
              uidx_v, nbidx_v, urows_v, nbrows_v, psum_v, sem_u, sem_nb):
    wid = lax.axis_index("s") * _NCORES + lax.axis_index("c")

    pltpu.sync_copy(user_hbm.at[pl.ds(wid * _UCP, _UCP)], uidx_v)
    pltpu.sync_copy(nearby_hbm.at[pl.ds(wid * _NCHP, _NCHP)], nbidx_v)

    for c in range(_UC):
        pltpu.async_copy(table_hbm.at[uidx_v.at[c]],
                         urows_v.at[pl.ds(c * _CH, _CH)], sem_u).wait()

    def _fire(c, carry):
        pltpu.async_copy(table_hbm.at[nbidx_v.at[c]],
                         nbrows_v.at[pl.ds(c * _CH, _CH)], sem_nb).wait()
        return carry
    lax.fori_loop(0, _NCH, _fire, 0)

    def _one(b, carry):
        s = nbrows_v[b * _L, :]
        for l in range(1, _L):
            s = s + nbrows_v[b * _L + l, :]
        p = s * urows_v[b, :]
        psum_v[b >> 3, pl.ds((b & 7) * _KP, _KP)] = p
        return carry
    lax.fori_loop(0, _BC, _one, 0)

    pltpu.sync_copy(psum_v, psum_hbm.at[pl.ds(wid * _PRT, _PRT)])


_BS = 256


def _tc_body(psum_ref, out_ref):
    rows = lax.broadcasted_iota(jnp.int32, (128, 8), 0)
    cols = lax.broadcasted_iota(jnp.int32, (128, 8), 1)
    bd = (rows // 16 == cols).astype(jnp.float32)
    out_ref[...] = jnp.dot(psum_ref[...], bd,
                           preferred_element_type=jnp.float32,
                           precision=lax.Precision.HIGHEST)


_tc_reduce = pl.pallas_call(
    _tc_body,
    grid=(_PR // _BS,),
    in_specs=[pl.BlockSpec((_BS, 128), lambda i: (i, 0))],
    out_specs=pl.BlockSpec((_BS, 8), lambda i: (i, 0)),
    out_shape=jax.ShapeDtypeStruct((_PR, 8), jnp.float32),
)


@jax.jit
def kernel(user, nearby, table):
    t16 = jnp.pad(table, ((0, _NROWS - table.shape[0]), (0, _KP - _K)))

    u32 = user.astype(jnp.int32)
    nb32 = nearby.astype(jnp.int32)
    u32 = jnp.where(u32 == 0, _ZROW, u32)
    nb32 = jnp.where(nb32 == 0, _ZROW, nb32)

    uidx = u32.reshape(_NW, _UC, _CH)
    uidx = jnp.pad(uidx, ((0, 0), (0, _UCP - _UC), (0, 0)))
    uidx = uidx.reshape(_NW * _UCP, _CH)

    nbidx = nb32.reshape(_NW, _NCH, _CH)
    nbidx = jnp.pad(nbidx, ((0, 0), (0, _NCHP - _NCH), (0, 0)))
    nbidx = nbidx.reshape(_NW * _NCHP, _CH)

    psum = _place_sc(uidx, nbidx, t16)
    return _tc_reduce(psum).reshape(_B)
